# baseline (device time: 91936 ns/iter reference)
import math

import jax
import jax.numpy as jnp
from jax import lax
from jax.experimental import pallas as pl
from jax.experimental.pallas import tpu as pltpu

N_DEV = 4
SQ = 1024
HQ = 8
DH = 128
BLK = SQ // N_DEV
SCALE = 0.08838834764831843


def _rope(t):
    pos = lax.broadcasted_iota(jnp.int32, t.shape, 0).astype(jnp.float32)
    d = lax.broadcasted_iota(jnp.int32, t.shape, 1)
    dk = ((d % DH) // 2) * 2
    inv = jnp.exp(dk.astype(jnp.float32) * (-math.log(10000.0) / DH))
    ang = pos * inv
    cos = jnp.cos(ang)
    sin = jnp.sin(ang)
    even = (d % 2) == 0
    t_rot = jnp.where(even, -jnp.roll(t, -1, axis=1), jnp.roll(t, 1, axis=1))
    return t * cos + t_rot * sin


def kernel(x, Wq, Wk, Wv, Wo):
    def body(x_ref, wq_ref, wk_ref, wv_ref, wo_ref, out_ref,
             q_scr, rs_send, rs_recv, ag_buf,
             rs_send_sems, rs_recv_sems, ag_send_sems, ag_recv_sems):
        my = lax.axis_index("i")
        left = (my - 1) % N_DEV
        right = (my + 1) % N_DEV

        barrier_sem = pltpu.get_barrier_semaphore()
        for nbr in [left, right]:
            pl.semaphore_signal(
                barrier_sem, inc=1,
                device_id=(nbr,), device_id_type=pl.DeviceIdType.MESH,
            )
        pl.semaphore_wait(barrier_sem, 2)

        xb = x_ref[0]
        wo = wo_ref[...]

        q = jnp.dot(xb, wq_ref[...], preferred_element_type=jnp.float32)
        k = jnp.dot(xb, wk_ref[...], preferred_element_type=jnp.float32)
        v = jnp.dot(xb, wv_ref[...],
                    preferred_element_type=jnp.float32).astype(jnp.bfloat16)

        q_scr[...] = (_rope(q) * SCALE).astype(jnp.bfloat16)
        k = _rope(k).astype(jnp.bfloat16)

        def attn_block(off):
            qb = q_scr[pl.ds(off, BLK), :]
            cols = []
            for h in range(HQ):
                sl = slice(h * DH, (h + 1) * DH)
                s = lax.dot_general(
                    qb[:, sl], k[:, sl], (((1,), (1,)), ((), ())),
                    preferred_element_type=jnp.float32,
                )
                e = jnp.exp(s)
                r = 1.0 / jnp.sum(e, axis=1, keepdims=True)
                c = jnp.dot(e.astype(jnp.bfloat16), v[:, sl],
                            preferred_element_type=jnp.float32) * r
                cols.append(c.astype(jnp.bfloat16))
            ctx = jnp.concatenate(cols, axis=1)
            return jnp.dot(ctx, wo, preferred_element_type=jnp.float32)

        rs = []
        p = attn_block(my * BLK)
        rs_send[0] = p.astype(jnp.bfloat16)
        rdma = pltpu.make_async_remote_copy(
            src_ref=rs_send.at[0], dst_ref=rs_recv.at[0],
            send_sem=rs_send_sems.at[0], recv_sem=rs_recv_sems.at[0],
            device_id=(right,), device_id_type=pl.DeviceIdType.MESH,
        )
        rdma.start()
        rs.append(rdma)

        val = None
        for j in range(1, N_DEV):
            b = (my - j) % N_DEV
            p = attn_block(b * BLK)
            rs[j - 1].wait_recv()
            val = p + rs_recv[j - 1].astype(jnp.float32)
            if j < N_DEV - 1:
                rs_send[j] = val.astype(jnp.bfloat16)
                rdma = pltpu.make_async_remote_copy(
                    src_ref=rs_send.at[j], dst_ref=rs_recv.at[j],
                    send_sem=rs_send_sems.at[j], recv_sem=rs_recv_sems.at[j],
                    device_id=(right,), device_id_type=pl.DeviceIdType.MESH,
                )
                rdma.start()
                rs.append(rdma)

        own = ((my + 1) % N_DEV) * BLK
        ag_buf[0] = val.astype(jnp.bfloat16)
        out_ref[0, pl.ds(own, BLK), :] = val

        ag_l = pltpu.make_async_remote_copy(
            src_ref=ag_buf.at[0], dst_ref=ag_buf.at[1],
            send_sem=ag_send_sems.at[0], recv_sem=ag_recv_sems.at[0],
            device_id=(left,), device_id_type=pl.DeviceIdType.MESH,
        )
        ag_r = pltpu.make_async_remote_copy(
            src_ref=ag_buf.at[0], dst_ref=ag_buf.at[2],
            send_sem=ag_send_sems.at[1], recv_sem=ag_recv_sems.at[1],
            device_id=(right,), device_id_type=pl.DeviceIdType.MESH,
        )
        ag_l.start()
        ag_r.start()

        ag_r.wait_recv()
        out_ref[0, pl.ds(my * BLK, BLK), :] = ag_buf[2].astype(jnp.float32)

        ag_l.wait_recv()
        out_ref[0, pl.ds(((my + 2) % N_DEV) * BLK, BLK), :] = (
            ag_buf[1].astype(jnp.float32)
        )

        ag_f = pltpu.make_async_remote_copy(
            src_ref=ag_buf.at[1], dst_ref=ag_buf.at[3],
            send_sem=ag_send_sems.at[2], recv_sem=ag_recv_sems.at[2],
            device_id=(left,), device_id_type=pl.DeviceIdType.MESH,
        )
        ag_f.start()
        ag_f.wait_recv()
        out_ref[0, pl.ds(((my + 3) % N_DEV) * BLK, BLK), :] = (
            ag_buf[3].astype(jnp.float32)
        )

        for r in rs:
            r.wait_send()
        ag_l.wait_send()
        ag_r.wait_send()
        ag_f.wait_send()

    out_shape = jax.ShapeDtypeStruct((1, SQ, 1024), jnp.float32)
    return pl.pallas_call(
        body,
        out_shape=out_shape,
        in_specs=[pl.BlockSpec(memory_space=pltpu.VMEM)] * 5,
        out_specs=pl.BlockSpec(memory_space=pltpu.VMEM),
        scratch_shapes=[
            pltpu.VMEM((SQ, HQ * DH), jnp.bfloat16),
            pltpu.VMEM((N_DEV - 1, BLK, 1024), jnp.bfloat16),
            pltpu.VMEM((N_DEV - 1, BLK, 1024), jnp.bfloat16),
            pltpu.VMEM((4, BLK, 1024), jnp.bfloat16),
            pltpu.SemaphoreType.DMA((N_DEV - 1,)),
            pltpu.SemaphoreType.DMA((N_DEV - 1,)),
            pltpu.SemaphoreType.DMA((3,)),
            pltpu.SemaphoreType.DMA((3,)),
        ],
        compiler_params=pltpu.CompilerParams(
            collective_id=0,
            vmem_limit_bytes=100 * 1024 * 1024,
        ),
    )(
        x.astype(jnp.bfloat16),
        Wq.astype(jnp.bfloat16),
        Wk.astype(jnp.bfloat16),
        Wv.astype(jnp.bfloat16),
        Wo.astype(jnp.bfloat16),
    )


# device time: 83965 ns/iter; 1.0949x vs baseline; 1.0949x over previous
import functools
import math

import jax
import jax.numpy as jnp
from jax import lax
from jax.experimental import pallas as pl
from jax.experimental.pallas import tpu as pltpu

N_DEV = 4
SQ = 1024
HQ = 8
DH = 128
R = 2
HALF = SQ // R
BLK = HALF // N_DEV
SCALE = 0.08838834764831843


def _cos_sin():
    pos = lax.broadcasted_iota(jnp.int32, (SQ, DH), 0).astype(jnp.float32)
    d = lax.broadcasted_iota(jnp.int32, (SQ, DH), 1)
    dk = (d // 2) * 2
    inv = jnp.exp(dk.astype(jnp.float32) * (-math.log(10000.0) / DH))
    ang = pos * inv
    cos = jnp.concatenate([jnp.cos(ang)] * HQ, axis=1)
    sin = jnp.concatenate([jnp.sin(ang)] * HQ, axis=1)
    return cos, sin


def _rope(t, cos, sin):
    even = (lax.broadcasted_iota(jnp.int32, t.shape, 1) % 2) == 0
    t_rot = jnp.where(even, -jnp.roll(t, -1, axis=1), jnp.roll(t, 1, axis=1))
    return t * cos + t_rot * sin


def kernel(x, Wq, Wk, Wv, Wo):
    def body(x_ref, wq_ref, wk_ref, wv_ref, wo_ref, out_ref,
             q_scr, rs_send, rs_recv, ag_buf,
             rs_send_sems, rs_recv_sems, ag_send_sems, ag_recv_sems):
        my = lax.axis_index("i")
        left = (my - 1) % N_DEV
        right = (my + 1) % N_DEV
        opp = (my + 2) % N_DEV

        barrier_sem = pltpu.get_barrier_semaphore()
        for nbr in [left, right]:
            pl.semaphore_signal(
                barrier_sem, inc=1,
                device_id=(nbr,), device_id_type=pl.DeviceIdType.MESH,
            )
        pl.semaphore_wait(barrier_sem, 2)

        xb = x_ref[0]
        wo = wo_ref[...]

        q = jnp.dot(xb, wq_ref[...], preferred_element_type=jnp.float32)
        k = jnp.dot(xb, wk_ref[...], preferred_element_type=jnp.float32)
        v = jnp.dot(xb, wv_ref[...],
                    preferred_element_type=jnp.float32).astype(jnp.bfloat16)

        cos, sin = _cos_sin()
        q_scr[...] = (_rope(q, cos, sin) * SCALE).astype(jnp.bfloat16)
        k = _rope(k, cos, sin).astype(jnp.bfloat16)

        def attn_block(off):
            qb = q_scr[pl.ds(off, BLK), :]
            cols = []
            for h in range(HQ):
                sl = slice(h * DH, (h + 1) * DH)
                s = lax.dot_general(
                    qb[:, sl], k[:, sl], (((1,), (1,)), ((), ())),
                    preferred_element_type=jnp.float32,
                )
                e = jnp.exp(s)
                r = 1.0 / jnp.sum(e, axis=1, keepdims=True)
                c = jnp.dot(e.astype(jnp.bfloat16), v[:, sl],
                            preferred_element_type=jnp.float32) * r
                cols.append(c.astype(jnp.bfloat16))
            ctx = jnp.concatenate(cols, axis=1)
            return jnp.dot(ctx, wo, preferred_element_type=jnp.float32)

        def rs_round(r):
            base = r * HALF
            descs = []
            p = attn_block(base + my * BLK)
            rs_send[r, 0] = p.astype(jnp.bfloat16)
            d = pltpu.make_async_remote_copy(
                src_ref=rs_send.at[r, 0], dst_ref=rs_recv.at[r, 0],
                send_sem=rs_send_sems.at[r, 0], recv_sem=rs_recv_sems.at[r, 0],
                device_id=(right,), device_id_type=pl.DeviceIdType.MESH,
            )
            d.start()
            descs.append(d)
            val = None
            for j in range(1, N_DEV):
                p = attn_block(base + ((my - j) % N_DEV) * BLK)
                descs[j - 1].wait_recv()
                val = p + rs_recv[r, j - 1].astype(jnp.float32)
                if j < N_DEV - 1:
                    rs_send[r, j] = val.astype(jnp.bfloat16)
                    d = pltpu.make_async_remote_copy(
                        src_ref=rs_send.at[r, j], dst_ref=rs_recv.at[r, j],
                        send_sem=rs_send_sems.at[r, j],
                        recv_sem=rs_recv_sems.at[r, j],
                        device_id=(right,), device_id_type=pl.DeviceIdType.MESH,
                    )
                    d.start()
                    descs.append(d)
            return descs, val

        AG_TARGETS = ((1,), (2,), (3,))

        def start_ag(r, val):
            base = r * HALF
            ag_buf[r, 0] = val.astype(jnp.bfloat16)
            out_ref[0, pl.ds(base + ((my + 1) % N_DEV) * BLK, BLK), :] = val
            descs = []
            for idx, dev in enumerate([left, right, opp]):
                d = pltpu.make_async_remote_copy(
                    src_ref=ag_buf.at[r, 0], dst_ref=ag_buf.at[r, idx + 1],
                    send_sem=ag_send_sems.at[r, idx],
                    recv_sem=ag_recv_sems.at[r, idx],
                    device_id=(dev,), device_id_type=pl.DeviceIdType.MESH,
                )
                d.start()
                descs.append(d)
            return descs

        def finish_ag(r, descs):
            base = r * HALF
            for idx, chunk in enumerate([opp, my, (my + 3) % N_DEV]):
                descs[idx].wait_recv()
                out_ref[0, pl.ds(base + chunk * BLK, BLK), :] = (
                    ag_buf[r, idx + 1].astype(jnp.float32)
                )

        rs0, val0 = rs_round(0)
        ag0 = start_ag(0, val0)
        rs1, val1 = rs_round(1)
        finish_ag(0, ag0)
        ag1 = start_ag(1, val1)
        finish_ag(1, ag1)

        for d in rs0 + rs1 + ag0 + ag1:
            d.wait_send()

        @functools.partial(
            pl.run_scoped, exit_sem=pltpu.SemaphoreType.REGULAR
        )
        def _(exit_sem):
            for nbr in [left, right]:
                pl.semaphore_signal(
                    exit_sem, inc=1,
                    device_id=(nbr,), device_id_type=pl.DeviceIdType.MESH,
                )
            pl.semaphore_wait(exit_sem, 2)

    out_shape = jax.ShapeDtypeStruct((1, SQ, 1024), jnp.float32)
    return pl.pallas_call(
        body,
        out_shape=out_shape,
        in_specs=[pl.BlockSpec(memory_space=pltpu.VMEM)] * 5,
        out_specs=pl.BlockSpec(memory_space=pltpu.VMEM),
        scratch_shapes=[
            pltpu.VMEM((SQ, HQ * DH), jnp.bfloat16),
            pltpu.VMEM((R, N_DEV - 1, BLK, 1024), jnp.bfloat16),
            pltpu.VMEM((R, N_DEV - 1, BLK, 1024), jnp.bfloat16),
            pltpu.VMEM((R, 4, BLK, 1024), jnp.bfloat16),
            pltpu.SemaphoreType.DMA((R, N_DEV - 1)),
            pltpu.SemaphoreType.DMA((R, N_DEV - 1)),
            pltpu.SemaphoreType.DMA((R, 3)),
            pltpu.SemaphoreType.DMA((R, 3)),
        ],
        compiler_params=pltpu.CompilerParams(
            collective_id=0,
            vmem_limit_bytes=100 * 1024 * 1024,
        ),
    )(
        x.astype(jnp.bfloat16),
        Wq.astype(jnp.bfloat16),
        Wk.astype(jnp.bfloat16),
        Wv.astype(jnp.bfloat16),
        Wo.astype(jnp.bfloat16),
    )


# device time: 83001 ns/iter; 1.1076x vs baseline; 1.0116x over previous
import functools
import math

import jax
import jax.numpy as jnp
from jax import lax
from jax.experimental import pallas as pl
from jax.experimental.pallas import tpu as pltpu

N_DEV = 4
SQ = 1024
HQ = 8
DH = 128
R = 2
HALF = SQ // R
BLK = HALF // N_DEV
SCALE = 0.08838834764831843


def _cos_sin():
    pos = lax.broadcasted_iota(jnp.int32, (SQ, DH), 0).astype(jnp.float32)
    d = lax.broadcasted_iota(jnp.int32, (SQ, DH), 1)
    dk = (d // 2) * 2
    inv = jnp.exp(dk.astype(jnp.float32) * (-math.log(10000.0) / DH))
    ang = pos * inv
    cos = jnp.concatenate([jnp.cos(ang)] * HQ, axis=1).astype(jnp.bfloat16)
    sin = jnp.concatenate([jnp.sin(ang)] * HQ, axis=1).astype(jnp.bfloat16)
    return cos, sin


def _rope(t, cos, sin):
    even = (lax.broadcasted_iota(jnp.int32, t.shape, 1) % 2) == 0
    t_rot = jnp.where(even, -jnp.roll(t, -1, axis=1), jnp.roll(t, 1, axis=1))
    return t * cos + t_rot * sin


def kernel(x, Wq, Wk, Wv, Wo):
    def body(x_ref, wq_ref, wk_ref, wv_ref, wo_ref, out_ref,
             q_scr, rs_send, rs_recv, ag_buf,
             rs_send_sems, rs_recv_sems, ag_send_sems, ag_recv_sems):
        my = lax.axis_index("i")
        left = (my - 1) % N_DEV
        right = (my + 1) % N_DEV
        opp = (my + 2) % N_DEV

        barrier_sem = pltpu.get_barrier_semaphore()
        for nbr in [left, right]:
            pl.semaphore_signal(
                barrier_sem, inc=1,
                device_id=(nbr,), device_id_type=pl.DeviceIdType.MESH,
            )
        pl.semaphore_wait(barrier_sem, 2)

        xb = x_ref[0]
        wo = wo_ref[...]

        q = jnp.dot(xb, wq_ref[...],
                    preferred_element_type=jnp.float32).astype(jnp.bfloat16)
        k = jnp.dot(xb, wk_ref[...],
                    preferred_element_type=jnp.float32).astype(jnp.bfloat16)
        v = jnp.dot(xb, wv_ref[...],
                    preferred_element_type=jnp.float32).astype(jnp.bfloat16)

        cos, sin = _cos_sin()
        q_scr[...] = _rope(q, cos, sin) * jnp.bfloat16(SCALE)
        k = _rope(k, cos, sin)

        def attn_block(off):
            qb = q_scr[pl.ds(off, BLK), :]
            cols = []
            for h in range(HQ):
                sl = slice(h * DH, (h + 1) * DH)
                s = lax.dot_general(
                    qb[:, sl], k[:, sl], (((1,), (1,)), ((), ())),
                    preferred_element_type=jnp.float32,
                ).astype(jnp.bfloat16)
                e = jnp.exp(s)
                r = 1.0 / jnp.sum(e.astype(jnp.float32), axis=1,
                                  keepdims=True)
                c = jnp.dot(e, v[:, sl],
                            preferred_element_type=jnp.float32) * r
                cols.append(c.astype(jnp.bfloat16))
            ctx = jnp.concatenate(cols, axis=1)
            return jnp.dot(ctx, wo, preferred_element_type=jnp.float32)

        def rs_round(r):
            base = r * HALF
            descs = []
            p = attn_block(base + my * BLK)
            rs_send[r, 0] = p.astype(jnp.bfloat16)
            d = pltpu.make_async_remote_copy(
                src_ref=rs_send.at[r, 0], dst_ref=rs_recv.at[r, 0],
                send_sem=rs_send_sems.at[r, 0], recv_sem=rs_recv_sems.at[r, 0],
                device_id=(right,), device_id_type=pl.DeviceIdType.MESH,
            )
            d.start()
            descs.append(d)
            val = None
            for j in range(1, N_DEV):
                p = attn_block(base + ((my - j) % N_DEV) * BLK)
                descs[j - 1].wait_recv()
                val = p + rs_recv[r, j - 1].astype(jnp.float32)
                if j < N_DEV - 1:
                    rs_send[r, j] = val.astype(jnp.bfloat16)
                    d = pltpu.make_async_remote_copy(
                        src_ref=rs_send.at[r, j], dst_ref=rs_recv.at[r, j],
                        send_sem=rs_send_sems.at[r, j],
                        recv_sem=rs_recv_sems.at[r, j],
                        device_id=(right,), device_id_type=pl.DeviceIdType.MESH,
                    )
                    d.start()
                    descs.append(d)
            return descs, val

        AG_TARGETS = ((1,), (2,), (3,))

        def start_ag(r, val):
            base = r * HALF
            ag_buf[r, 0] = val.astype(jnp.bfloat16)
            out_ref[0, pl.ds(base + ((my + 1) % N_DEV) * BLK, BLK), :] = val
            descs = []
            for idx, dev in enumerate([left, right, opp]):
                d = pltpu.make_async_remote_copy(
                    src_ref=ag_buf.at[r, 0], dst_ref=ag_buf.at[r, idx + 1],
                    send_sem=ag_send_sems.at[r, idx],
                    recv_sem=ag_recv_sems.at[r, idx],
                    device_id=(dev,), device_id_type=pl.DeviceIdType.MESH,
                )
                d.start()
                descs.append(d)
            return descs

        def finish_ag(r, descs):
            base = r * HALF
            for idx, chunk in enumerate([opp, my, (my + 3) % N_DEV]):
                descs[idx].wait_recv()
                out_ref[0, pl.ds(base + chunk * BLK, BLK), :] = (
                    ag_buf[r, idx + 1].astype(jnp.float32)
                )

        rs0, val0 = rs_round(0)
        ag0 = start_ag(0, val0)
        rs1, val1 = rs_round(1)
        finish_ag(0, ag0)
        ag1 = start_ag(1, val1)
        finish_ag(1, ag1)

        for d in rs0 + rs1 + ag0 + ag1:
            d.wait_send()

        @functools.partial(
            pl.run_scoped, exit_sem=pltpu.SemaphoreType.REGULAR
        )
        def _(exit_sem):
            for nbr in [left, right]:
                pl.semaphore_signal(
                    exit_sem, inc=1,
                    device_id=(nbr,), device_id_type=pl.DeviceIdType.MESH,
                )
            pl.semaphore_wait(exit_sem, 2)

    out_shape = jax.ShapeDtypeStruct((1, SQ, 1024), jnp.float32)
    return pl.pallas_call(
        body,
        out_shape=out_shape,
        in_specs=[pl.BlockSpec(memory_space=pltpu.VMEM)] * 5,
        out_specs=pl.BlockSpec(memory_space=pltpu.VMEM),
        scratch_shapes=[
            pltpu.VMEM((SQ, HQ * DH), jnp.bfloat16),
            pltpu.VMEM((R, N_DEV - 1, BLK, 1024), jnp.bfloat16),
            pltpu.VMEM((R, N_DEV - 1, BLK, 1024), jnp.bfloat16),
            pltpu.VMEM((R, 4, BLK, 1024), jnp.bfloat16),
            pltpu.SemaphoreType.DMA((R, N_DEV - 1)),
            pltpu.SemaphoreType.DMA((R, N_DEV - 1)),
            pltpu.SemaphoreType.DMA((R, 3)),
            pltpu.SemaphoreType.DMA((R, 3)),
        ],
        compiler_params=pltpu.CompilerParams(
            collective_id=0,
            vmem_limit_bytes=100 * 1024 * 1024,
        ),
    )(
        x.astype(jnp.bfloat16),
        Wq.astype(jnp.bfloat16),
        Wk.astype(jnp.bfloat16),
        Wv.astype(jnp.bfloat16),
        Wo.astype(jnp.bfloat16),
    )


# device time: 61961 ns/iter; 1.4838x vs baseline; 1.3396x over previous
import functools
import math

import jax
import jax.numpy as jnp
from jax import lax
from jax.experimental import pallas as pl
from jax.experimental.pallas import tpu as pltpu

N_DEV = 4
SQ = 1024
HQ = 8
DH = 128
R = 1
HALF = SQ // R
BLK = HALF // N_DEV
SCALE = 0.08838834764831843


def _cos_sin():
    pos = lax.broadcasted_iota(jnp.int32, (SQ, DH), 0).astype(jnp.float32)
    d = lax.broadcasted_iota(jnp.int32, (SQ, DH), 1)
    dk = (d // 2) * 2
    inv = jnp.exp(dk.astype(jnp.float32) * (-math.log(10000.0) / DH))
    ang = pos * inv
    cos = jnp.concatenate([jnp.cos(ang)] * HQ, axis=1).astype(jnp.bfloat16)
    sin = jnp.concatenate([jnp.sin(ang)] * HQ, axis=1).astype(jnp.bfloat16)
    return cos, sin


def _rope(t, cos, sin):
    even = (lax.broadcasted_iota(jnp.int32, t.shape, 1) % 2) == 0
    t_rot = jnp.where(even, -jnp.roll(t, -1, axis=1), jnp.roll(t, 1, axis=1))
    return t * cos + t_rot * sin


def kernel(x, Wq, Wk, Wv, Wo):
    def body(x_ref, wq_ref, wk_ref, wv_ref, wo_ref, out_ref,
             q_scr, rs_send, rs_recv, ag_buf,
             rs_send_sems, rs_recv_sems, ag_send_sems, ag_recv_sems):
        my = lax.axis_index("i")
        left = (my - 1) % N_DEV
        right = (my + 1) % N_DEV
        opp = (my + 2) % N_DEV

        barrier_sem = pltpu.get_barrier_semaphore()
        for nbr in [left, right]:
            pl.semaphore_signal(
                barrier_sem, inc=1,
                device_id=(nbr,), device_id_type=pl.DeviceIdType.MESH,
            )

        xb = x_ref[0].astype(jnp.bfloat16)
        wo = wo_ref[...].astype(jnp.bfloat16)

        q = jnp.dot(xb, wq_ref[...].astype(jnp.bfloat16),
                    preferred_element_type=jnp.float32).astype(jnp.bfloat16)
        k = jnp.dot(xb, wk_ref[...].astype(jnp.bfloat16),
                    preferred_element_type=jnp.float32).astype(jnp.bfloat16)
        v = jnp.dot(xb, wv_ref[...].astype(jnp.bfloat16),
                    preferred_element_type=jnp.float32).astype(jnp.bfloat16)

        cos, sin = _cos_sin()
        q_scr[...] = _rope(q, cos, sin) * jnp.bfloat16(SCALE)
        k = _rope(k, cos, sin)

        def attn_block(off):
            qb = q_scr[pl.ds(off, BLK), :]
            cols = []
            for h in range(HQ):
                sl = slice(h * DH, (h + 1) * DH)
                s = lax.dot_general(
                    qb[:, sl], k[:, sl], (((1,), (1,)), ((), ())),
                    preferred_element_type=jnp.float32,
                ).astype(jnp.bfloat16)
                e = jnp.exp(s)
                r = 1.0 / jnp.sum(e.astype(jnp.float32), axis=1,
                                  keepdims=True)
                c = jnp.dot(e, v[:, sl],
                            preferred_element_type=jnp.float32) * r
                cols.append(c.astype(jnp.bfloat16))
            ctx = jnp.concatenate(cols, axis=1)
            return jnp.dot(ctx, wo, preferred_element_type=jnp.float32)

        def rs_round(r):
            base = r * HALF
            descs = []
            p = attn_block(base + my * BLK)
            rs_send[r, 0] = p.astype(jnp.bfloat16)
            if r == 0:
                pl.semaphore_wait(barrier_sem, 2)
            d = pltpu.make_async_remote_copy(
                src_ref=rs_send.at[r, 0], dst_ref=rs_recv.at[r, 0],
                send_sem=rs_send_sems.at[r, 0], recv_sem=rs_recv_sems.at[r, 0],
                device_id=(right,), device_id_type=pl.DeviceIdType.MESH,
            )
            d.start()
            descs.append(d)
            val = None
            for j in range(1, N_DEV):
                p = attn_block(base + ((my - j) % N_DEV) * BLK)
                descs[j - 1].wait_recv()
                val = p.astype(jnp.bfloat16) + rs_recv[r, j - 1]
                if j < N_DEV - 1:
                    rs_send[r, j] = val
                    d = pltpu.make_async_remote_copy(
                        src_ref=rs_send.at[r, j], dst_ref=rs_recv.at[r, j],
                        send_sem=rs_send_sems.at[r, j],
                        recv_sem=rs_recv_sems.at[r, j],
                        device_id=(right,), device_id_type=pl.DeviceIdType.MESH,
                    )
                    d.start()
                    descs.append(d)
            return descs, val

        AG_TARGETS = ((1,), (2,), (3,))

        def start_ag(r, val):
            base = r * HALF
            ag_buf[r, 0] = val
            out_ref[0, pl.ds(base + ((my + 1) % N_DEV) * BLK, BLK), :] = val
            descs = []
            for idx, dev in enumerate([left, right, opp]):
                d = pltpu.make_async_remote_copy(
                    src_ref=ag_buf.at[r, 0], dst_ref=ag_buf.at[r, idx + 1],
                    send_sem=ag_send_sems.at[r, idx],
                    recv_sem=ag_recv_sems.at[r, idx],
                    device_id=(dev,), device_id_type=pl.DeviceIdType.MESH,
                )
                d.start()
                descs.append(d)
            return descs

        def finish_ag(r, descs):
            base = r * HALF
            for idx, chunk in enumerate([opp, my, (my + 3) % N_DEV]):
                descs[idx].wait_recv()
                out_ref[0, pl.ds(base + chunk * BLK, BLK), :] = (
                    ag_buf[r, idx + 1]
                )

        all_descs = []
        prev_ag = None
        for r in range(R):
            descs, val = rs_round(r)
            all_descs += descs
            if prev_ag is not None:
                finish_ag(r - 1, prev_ag)
            prev_ag = start_ag(r, val)
            all_descs += prev_ag
        finish_ag(R - 1, prev_ag)

        for d in all_descs:
            d.wait_send()

        @functools.partial(
            pl.run_scoped, exit_sem=pltpu.SemaphoreType.REGULAR
        )
        def _(exit_sem):
            for nbr in [left, right]:
                pl.semaphore_signal(
                    exit_sem, inc=1,
                    device_id=(nbr,), device_id_type=pl.DeviceIdType.MESH,
                )
            pl.semaphore_wait(exit_sem, 2)

    out_shape = jax.ShapeDtypeStruct((1, SQ, 1024), jnp.bfloat16)
    return pl.pallas_call(
        body,
        out_shape=out_shape,
        in_specs=[pl.BlockSpec(memory_space=pltpu.VMEM)] * 5,
        out_specs=pl.BlockSpec(memory_space=pltpu.VMEM),
        scratch_shapes=[
            pltpu.VMEM((SQ, HQ * DH), jnp.bfloat16),
            pltpu.VMEM((R, N_DEV - 1, BLK, 1024), jnp.bfloat16),
            pltpu.VMEM((R, N_DEV - 1, BLK, 1024), jnp.bfloat16),
            pltpu.VMEM((R, 4, BLK, 1024), jnp.bfloat16),
            pltpu.SemaphoreType.DMA((R, N_DEV - 1)),
            pltpu.SemaphoreType.DMA((R, N_DEV - 1)),
            pltpu.SemaphoreType.DMA((R, 3)),
            pltpu.SemaphoreType.DMA((R, 3)),
        ],
        compiler_params=pltpu.CompilerParams(
            collective_id=0,
            vmem_limit_bytes=100 * 1024 * 1024,
        ),
    )(x, Wq, Wk, Wv, Wo)
